# Initial kernel scaffold; baseline (speedup 1.0000x reference)
import jax
import jax.numpy as jnp
from jax.experimental import pallas as pl


def kernel(inputs, E0, E1, W2, b2, W3, b3, W4, b4, W5, b5, W6, b6, W7, b7):
    raise NotImplementedError("write your pallas kernel here")


# speedup vs baseline: 1.5141x; 1.5141x over previous
import functools

import jax
import jax.numpy as jnp
from jax.experimental import pallas as pl
from jax.experimental.pallas import tpu as pltpu
from jax.experimental.pallas import tpu_sc as plsc

_B, _W, _D = 1024, 200, 64
_HIST, _FUT = 150, 50
_V1 = 1000
_HC = 7
_FC = 3
_HIST_CH = [7, None, 5, 6, 2, 3, 4]
_FUT_CH = [None, 5, 6]

_NC, _NS = 2, 16
_NW = _NC * _NS


def _tc_body(x_ref, m_ref, bias_ref, mf_ref, biasf_ref, e1_ref, hist_ref, fut_ref):
    x = x_ref[0]
    xh = x[:_HIST, :]
    xf = x[_HIST:, :]
    dh = jnp.dot(xh, m_ref[...], preferred_element_type=jnp.float32,
                 precision=jax.lax.Precision.HIGHEST) + bias_ref[...]
    df = jnp.dot(xf, mf_ref[...], preferred_element_type=jnp.float32,
                 precision=jax.lax.Precision.HIGHEST) + biasf_ref[...]
    ih = xh[:, 1].astype(jnp.int32)
    if_ = xf[:, 1].astype(jnp.int32)
    ohh = (ih[:, None] == jax.lax.broadcasted_iota(jnp.int32, (_HIST, _V1), 1)).astype(jnp.bfloat16)
    ohf = (if_[:, None] == jax.lax.broadcasted_iota(jnp.int32, (_FUT, _V1), 1)).astype(jnp.bfloat16)
    e1h = jnp.dot(ohh, e1_ref[...], preferred_element_type=jnp.float32)
    e1f = jnp.dot(ohf, e1_ref[...], preferred_element_type=jnp.float32)
    hist_ref[...] = dh.reshape(1, _HIST, _HC * _D)
    hist_ref[0, :, 1 * _D:2 * _D] = e1h
    fut_ref[...] = df.reshape(1, _FUT, _FC * _D)
    fut_ref[0, :, 0:_D] = e1f


def _dense_outputs(inputs, e1_bf, m, bias, mf, biasf):
    return pl.pallas_call(
        _tc_body,
        grid=(_B,),
        in_specs=[
            pl.BlockSpec((1, _W, 8), lambda i: (i, 0, 0)),
            pl.BlockSpec((8, _HC * _D), lambda i: (0, 0)),
            pl.BlockSpec((1, _HC * _D), lambda i: (0, 0)),
            pl.BlockSpec((8, _FC * _D), lambda i: (0, 0)),
            pl.BlockSpec((1, _FC * _D), lambda i: (0, 0)),
            pl.BlockSpec((_V1, _D), lambda i: (0, 0)),
        ],
        out_specs=[
            pl.BlockSpec((1, _HIST, _HC * _D), lambda i: (i, 0, 0)),
            pl.BlockSpec((1, _FUT, _FC * _D), lambda i: (i, 0, 0)),
        ],
        out_shape=[
            jax.ShapeDtypeStruct((_B, _HIST, _HC * _D), jnp.float32),
            jax.ShapeDtypeStruct((_B, _FUT, _FC * _D), jnp.float32),
        ],
    )(inputs, m, bias, mf, biasf, e1_bf)


def _static_gather(idx0, E0):
    bpw = _B // _NW
    mesh = plsc.VectorSubcoreMesh(core_axis_name="c", subcore_axis_name="s")

    @functools.partial(
        pl.kernel,
        mesh=mesh,
        out_type=jax.ShapeDtypeStruct((_B, _D), jnp.float32),
        compiler_params=pltpu.CompilerParams(use_tc_tiling_on_sc=False),
        scratch_types=[
            pltpu.VMEM((bpw,), jnp.int32),
            pltpu.VMEM((bpw, _D), jnp.float32),
            pltpu.SemaphoreType.DMA,
        ],
    )
    def k(idx_hbm, table_hbm, out_hbm, idx_v, rows_v, sem):
        wid = jax.lax.axis_index("s") * _NC + jax.lax.axis_index("c")
        base = wid * bpw
        pltpu.sync_copy(idx_hbm.at[pl.ds(base, bpw)], idx_v)
        pltpu.async_copy(table_hbm.at[idx_v], rows_v, sem).wait()
        pltpu.sync_copy(rows_v, out_hbm.at[pl.ds(base, bpw)])

    return k(idx0, E0)


def kernel(inputs, E0, E1, W2, b2, W3, b3, W4, b4, W5, b5, W6, b6, W7, b7):
    ws = {2: (W2, b2), 3: (W3, b3), 4: (W4, b4), 5: (W5, b5), 6: (W6, b6), 7: (W7, b7)}

    def proj(chans):
        mcols, bcols = [], []
        for v in chans:
            if v is None:
                mcols.append(jnp.zeros((8, _D), jnp.float32))
                bcols.append(jnp.zeros((_D,), jnp.float32))
            else:
                wv, bv = ws[v]
                mcols.append(jnp.zeros((8, _D), jnp.float32).at[v].set(wv[0]))
                bcols.append(bv)
        return jnp.concatenate(mcols, axis=1), jnp.concatenate(bcols)[None, :]

    m, bias = proj(_HIST_CH)
    mf, biasf = proj(_FUT_CH)
    e1_bf = E1.astype(jnp.bfloat16)

    hist, fut = _dense_outputs(inputs, e1_bf, m, bias, mf, biasf)
    idx0 = inputs[:, 0, 0].astype(jnp.int32)
    static = _static_gather(idx0, E0)

    return (
        static.reshape(_B, 1, _D),
        hist.reshape(_B, _HIST, _HC, _D),
        fut.reshape(_B, _FUT, _FC, _D),
    )
